# baseline (speedup 1.0000x reference)
import functools

import jax
import jax.numpy as jnp
from jax import lax
from jax.experimental import pallas as pl
from jax.experimental.pallas import tpu as pltpu
from jax.experimental.pallas import tpu_sc as plsc

_INFO = plsc.get_sparse_core_info()
_NC, _NS = _INFO.num_cores, _INFO.num_subcores
_NW = _NC * _NS
_CH = 128


def _build(N, V, D):
    n_chunks = N // _CH
    pw = n_chunks // _NW
    mesh = plsc.VectorSubcoreMesh(core_axis_name="c", subcore_axis_name="s")

    @functools.partial(
        pl.kernel,
        out_type=jax.ShapeDtypeStruct((N, D), jnp.float32),
        mesh=mesh,
        compiler_params=pltpu.CompilerParams(use_tc_tiling_on_sc=False),
        scratch_types=[
            pltpu.VMEM((pw, _CH), jnp.int32),
            pltpu.VMEM((pw, _CH), jnp.int32),
            pltpu.VMEM((_CH, D), jnp.float32),
            pltpu.VMEM((_CH, D), jnp.float32),
            pltpu.SemaphoreType.DMA,
            pltpu.SemaphoreType.DMA,
            pltpu.SemaphoreType.DMA,
        ],
    )
    def two_level_gather(batch_hbm, gmap_hbm, emb_hbm, out_hbm,
                         idx_v, gids_v, rows0, rows1, sem_g, sem0, sem1):
        wid = lax.axis_index("s") * _NC + lax.axis_index("c")
        row_base = wid * pw

        pltpu.sync_copy(batch_hbm.at[wid], idx_v)

        def fire_remap(j, carry):
            pltpu.async_copy(gmap_hbm.at[idx_v.at[j]], gids_v.at[j], sem_g)
            return carry
        lax.fori_loop(0, pw, fire_remap, 0)
        pltpu.make_async_copy(batch_hbm.at[wid], gids_v, sem_g).wait()

        rows = (rows0, rows1)
        sems = (sem0, sem1)

        pltpu.async_copy(emb_hbm.at[gids_v.at[0]], rows0, sem0)
        pltpu.async_copy(emb_hbm.at[gids_v.at[1]], rows1, sem1)

        def body(g, carry):
            for b in range(2):
                j = 2 * g + b
                pltpu.make_async_copy(emb_hbm.at[gids_v.at[j]], rows[b],
                                      sems[b]).wait()
                pltpu.sync_copy(rows[b],
                                out_hbm.at[pl.ds((row_base + j) * _CH, _CH)])
                pltpu.async_copy(emb_hbm.at[gids_v.at[j + 2]], rows[b],
                                 sems[b])
            return carry
        lax.fori_loop(0, pw // 2 - 1, body, 0)

        for b in range(2):
            j = pw - 2 + b
            pltpu.make_async_copy(emb_hbm.at[gids_v.at[j]], rows[b],
                                  sems[b]).wait()
            pltpu.sync_copy(rows[b],
                            out_hbm.at[pl.ds((row_base + j) * _CH, _CH)])

    return two_level_gather


def kernel(batch, glove_id_map, embeddings):
    B, L = batch.shape
    V, D = embeddings.shape
    N = B * L
    idx2d = batch.reshape(_NW, N // (_NW * _CH), _CH).astype(jnp.int32)
    flat = _build(N, V, D)(idx2d, glove_id_map, embeddings)
    return flat.reshape(B, L, D)


# speedup vs baseline: 1.0152x; 1.0152x over previous
import functools

import jax
import jax.numpy as jnp
from jax import lax
from jax.experimental import pallas as pl
from jax.experimental.pallas import tpu as pltpu
from jax.experimental.pallas import tpu_sc as plsc

_INFO = plsc.get_sparse_core_info()
_NC, _NS = _INFO.num_cores, _INFO.num_subcores
_NW = _NC * _NS
_CH = 128
_DP = 256


def _transpose_table(et, V, D):
    BV = 640

    def body(in_ref, out_ref):
        x = in_ref[...]
        eye = (lax.broadcasted_iota(jnp.int32, (D, D), 0)
               == lax.broadcasted_iota(jnp.int32, (D, D), 1)
               ).astype(jnp.float32)
        xt = lax.dot_general(x, eye, (((0,), (0,)), ((), ())),
                             preferred_element_type=jnp.float32)
        out_ref[:, :D] = xt

    return pl.pallas_call(
        body,
        grid=(V // BV,),
        in_specs=[pl.BlockSpec((D, BV), lambda i: (0, i))],
        out_specs=pl.BlockSpec((BV, _DP), lambda i: (i, 0)),
        out_shape=jax.ShapeDtypeStruct((V, _DP), jnp.float32),
    )(et)


def _transpose_out(rows, L, B, D):
    BB = 256

    def body(in_ref, out_ref):
        x = in_ref[0]
        eye = (lax.broadcasted_iota(jnp.int32, (BB, BB), 0)
               == lax.broadcasted_iota(jnp.int32, (BB, BB), 1)
               ).astype(jnp.float32)
        xt = lax.dot_general(x, eye, (((0,), (0,)), ((), ())),
                             preferred_element_type=jnp.float32)
        out_ref[0] = xt[:D]

    return pl.pallas_call(
        body,
        grid=(L, B // BB),
        in_specs=[pl.BlockSpec((1, BB, _DP), lambda l, i: (l, i, 0))],
        out_specs=pl.BlockSpec((1, D, BB), lambda l, i: (l, 0, i)),
        out_shape=jax.ShapeDtypeStruct((L, D, B), jnp.float32),
    )(rows)


def _build_gather(N, V):
    n_chunks = N // _CH
    pw = n_chunks // _NW
    mesh = plsc.VectorSubcoreMesh(core_axis_name="c", subcore_axis_name="s")

    @functools.partial(
        pl.kernel,
        out_type=jax.ShapeDtypeStruct((N, _DP), jnp.float32),
        mesh=mesh,
        compiler_params=pltpu.CompilerParams(use_tc_tiling_on_sc=True),
        scratch_types=[
            pltpu.VMEM((pw, _CH), jnp.int32),
            pltpu.VMEM((pw, _CH), jnp.int32),
            pltpu.VMEM((_CH, _DP), jnp.float32),
            pltpu.VMEM((_CH, _DP), jnp.float32),
            pltpu.SemaphoreType.DMA,
            pltpu.SemaphoreType.DMA,
            pltpu.SemaphoreType.DMA,
        ],
    )
    def two_level_gather(batch_hbm, gmap_hbm, emb_hbm, out_hbm,
                         idx_v, gids_v, rows0, rows1, sem_g, sem0, sem1):
        wid = lax.axis_index("s") * _NC + lax.axis_index("c")
        row_base = wid * pw

        pltpu.sync_copy(batch_hbm.at[wid], idx_v)

        def fire_remap(j, carry):
            pltpu.async_copy(gmap_hbm.at[idx_v.at[j]], gids_v.at[j], sem_g)
            return carry
        lax.fori_loop(0, pw, fire_remap, 0)
        pltpu.make_async_copy(batch_hbm.at[wid], gids_v, sem_g).wait()

        rows = (rows0, rows1)
        sems = (sem0, sem1)

        pltpu.async_copy(emb_hbm.at[gids_v.at[0]], rows0, sem0)
        pltpu.async_copy(emb_hbm.at[gids_v.at[1]], rows1, sem1)

        def body(g, carry):
            for b in range(2):
                j = 2 * g + b
                pltpu.make_async_copy(emb_hbm.at[gids_v.at[j]], rows[b],
                                      sems[b]).wait()
                pltpu.sync_copy(rows[b],
                                out_hbm.at[pl.ds((row_base + j) * _CH, _CH)])
                pltpu.async_copy(emb_hbm.at[gids_v.at[j + 2]], rows[b],
                                 sems[b])
            return carry
        lax.fori_loop(0, pw // 2 - 1, body, 0)

        for b in range(2):
            j = pw - 2 + b
            pltpu.make_async_copy(emb_hbm.at[gids_v.at[j]], rows[b],
                                  sems[b]).wait()
            pltpu.sync_copy(rows[b],
                            out_hbm.at[pl.ds((row_base + j) * _CH, _CH)])

    return two_level_gather


def kernel(batch, glove_id_map, embeddings):
    B, L = batch.shape
    V, D = embeddings.shape
    N = B * L
    idx3d = batch.T.reshape(_NW, N // (_NW * _CH), _CH).astype(jnp.int32)
    e4 = _transpose_table(embeddings.T, V, D)
    rows = _build_gather(N, V)(idx3d, glove_id_map, e4)
    out_t = _transpose_out(rows.reshape(L, B, _DP), L, B, D)
    return jnp.transpose(out_t, (2, 0, 1))
